# baseline (speedup 1.0000x reference)
import functools

import jax
import jax.numpy as jnp
from jax import lax
from jax.experimental import pallas as pl
from jax.experimental.pallas import tpu as pltpu
from jax.experimental.pallas import tpu_sc as plsc

B, S, D = 8, 512, 768
SPAN_NUM = 2048
MAX_W = 4
D_FF = 3072
N_CLS = 3
BN = 512
G = (B * SPAN_NUM) // BN
_SQRT_HALF = 0.7071067811865476

NC_SC, NS_SC = 2, 16
NW = NC_SC * NS_SC
SPW = (B * SPAN_NUM) // NW
CH = 32
ROWS = MAX_W * CH
NCHUNK = SPW // CH
ZROW = B * S

_sc_mesh = plsc.VectorSubcoreMesh(core_axis_name="c", subcore_axis_name="s")


@functools.partial(
    pl.kernel,
    mesh=_sc_mesh,
    out_type=jax.ShapeDtypeStruct((B * SPAN_NUM, D), jnp.float32),
    scratch_types=[
        pltpu.VMEM((ROWS,), jnp.int32),
        pltpu.VMEM((ROWS, D), jnp.float32),
        pltpu.VMEM((CH, D), jnp.float32),
        pltpu.SemaphoreType.DMA,
    ],
)
def _sc_span_sum(xpad, idx, out, idx_v, rows_v, emb_v, sem):
    wid = lax.axis_index("s") * NC_SC + lax.axis_index("c")
    base = wid * SPW

    def do_chunk(ci, carry):
        sb = base + ci * CH
        pltpu.sync_copy(idx.at[pl.ds(sb * MAX_W, ROWS)], idx_v)
        pltpu.async_copy(xpad.at[idx_v], rows_v, sem).wait()

        def do_span(c, carry2):
            for j in range(D // 16):
                sl = pl.ds(j * 16, 16)
                emb_v[c, sl] = (rows_v[4 * c, sl] + rows_v[4 * c + 1, sl]
                                + rows_v[4 * c + 2, sl] + rows_v[4 * c + 3, sl])
            return carry2

        lax.fori_loop(0, CH, do_span, 0)
        pltpu.sync_copy(emb_v, out.at[pl.ds(sb, CH)])
        return carry

    lax.fori_loop(0, NCHUNK, do_chunk, 0)


def _ffn_ln(emb, wi_ref, bi_ref, wo_ref, bo_ref, g_ref, be_ref):
    inter = jnp.dot(emb, wi_ref[...], preferred_element_type=jnp.float32)
    inter = inter + bi_ref[...]
    inter = 0.5 * inter * (1.0 + lax.erf(inter * _SQRT_HALF))
    out = jnp.dot(inter, wo_ref[...], preferred_element_type=jnp.float32)
    out = out + bo_ref[...] + emb
    m = jnp.mean(out, axis=-1, keepdims=True)
    d = out - m
    v = jnp.mean(d * d, axis=-1, keepdims=True)
    return d * lax.rsqrt(v + 1e-12) * g_ref[...] + be_ref[...]


def _body(es_ref, iw_ref,
          wi_f_ref, bi_f_ref, wo_f_ref, bo_f_ref, g_f_ref, be_f_ref,
          wi_r_ref, bi_r_ref, wo_r_ref, bo_r_ref, g_r_ref, be_r_ref,
          wa_ref, ba_ref, wop_ref, bop_ref, out_ref):
    emb = es_ref[0] * iw_ref[0]
    y1 = _ffn_ln(emb, wi_f_ref, bi_f_ref, wo_f_ref, bo_f_ref, g_f_ref, be_f_ref)
    y2 = _ffn_ln(emb, wi_r_ref, bi_r_ref, wo_r_ref, bo_r_ref, g_r_ref, be_r_ref)
    c1 = jnp.dot(y1, wa_ref[...], preferred_element_type=jnp.float32) + ba_ref[...]
    c2 = jnp.dot(y2, wop_ref[...], preferred_element_type=jnp.float32) + bop_ref[...]
    out_ref[0] = jnp.concatenate([c1, c2], axis=-1)


@jax.jit
def _run(embsum, invw,
         Wi_f, bi_f, Wo_f, bo_f, g_f, be_f,
         Wi_r, bi_r, Wo_r, bo_r, g_r, be_r, Wa, ba, Wop, bop):
    const2 = pl.BlockSpec((1, D_FF), lambda i: (0, 0))
    constd = pl.BlockSpec((1, D), lambda i: (0, 0))
    w_big = pl.BlockSpec((D, D_FF), lambda i: (0, 0))
    w_big_t = pl.BlockSpec((D_FF, D), lambda i: (0, 0))
    w_cls = pl.BlockSpec((D, N_CLS), lambda i: (0, 0))
    b_cls = pl.BlockSpec((1, N_CLS), lambda i: (0, 0))
    out = pl.pallas_call(
        _body,
        grid=(G,),
        in_specs=[
            pl.BlockSpec((1, BN, D), lambda i: (i, 0, 0)),
            pl.BlockSpec((1, BN, 1), lambda i: (i, 0, 0)),
            w_big, const2, w_big_t, constd, constd, constd,
            w_big, const2, w_big_t, constd, constd, constd,
            w_cls, b_cls, w_cls, b_cls,
        ],
        out_specs=pl.BlockSpec((1, BN, 2 * N_CLS), lambda i: (i, 0, 0)),
        out_shape=jax.ShapeDtypeStruct((G, BN, 2 * N_CLS), jnp.float32),
        compiler_params=pltpu.CompilerParams(
            dimension_semantics=("arbitrary",),
            vmem_limit_bytes=120 * 1024 * 1024,
        ),
    )(embsum, invw,
      Wi_f, bi_f.reshape(1, D_FF), Wo_f, bo_f.reshape(1, D),
      g_f.reshape(1, D), be_f.reshape(1, D),
      Wi_r, bi_r.reshape(1, D_FF), Wo_r, bo_r.reshape(1, D),
      g_r.reshape(1, D), be_r.reshape(1, D),
      Wa, ba.reshape(1, N_CLS), Wop, bop.reshape(1, N_CLS))
    return out.reshape(B, SPAN_NUM, 2 * N_CLS)


def kernel(input_bert_features, attention_mask, spans, span_mask,
           related_spans_tensor, sentence_length,
           Wi_f, bi_f, Wo_f, bo_f, g_f, be_f,
           Wi_r, bi_r, Wo_r, bo_r, g_r, be_r, Wa, ba, Wop, bop):
    start = spans[..., 0]
    width = spans[..., 2]
    offs = jnp.arange(MAX_W, dtype=jnp.int32)
    brange = jnp.arange(B, dtype=jnp.int32) * S
    tok = jnp.clip(start[..., None] + offs, 0, S - 1) + brange[:, None, None]
    idx = jnp.where(offs < width[..., None], tok, ZROW)
    xpad = jnp.concatenate(
        [input_bert_features.reshape(B * S, D),
         jnp.zeros((8, D), jnp.float32)], axis=0)
    embsum = _sc_span_sum(xpad, idx.reshape(-1))

    invw = (1.0 / jnp.maximum(width.astype(jnp.float32), 1.0))
    invw = invw * span_mask.astype(jnp.float32)
    return _run(embsum.reshape(G, BN, D), invw.reshape(G, BN, 1),
                Wi_f, bi_f, Wo_f, bo_f, g_f, be_f,
                Wi_r, bi_r, Wo_r, bo_r, g_r, be_r, Wa, ba, Wop, bop)


# speedup vs baseline: 1.0037x; 1.0037x over previous
import functools

import jax
import jax.numpy as jnp
from jax import lax
from jax.experimental import pallas as pl
from jax.experimental.pallas import tpu as pltpu
from jax.experimental.pallas import tpu_sc as plsc

B, S, D = 8, 512, 768
SPAN_NUM = 2048
MAX_W = 4
D_FF = 3072
N_CLS = 3
BN = 512
G = (B * SPAN_NUM) // BN
_SQRT_HALF = 0.7071067811865476

NC_SC, NS_SC = 2, 16
NW = NC_SC * NS_SC
SPW = (B * SPAN_NUM) // NW
CH = 16
ROWS = MAX_W * CH
NCHUNK = SPW // CH
ZROW = B * S

_sc_mesh = plsc.VectorSubcoreMesh(core_axis_name="c", subcore_axis_name="s")


@functools.partial(
    pl.kernel,
    mesh=_sc_mesh,
    out_type=jax.ShapeDtypeStruct((B * SPAN_NUM, D), jnp.float32),
    scratch_types=[
        pltpu.VMEM((SPW * MAX_W,), jnp.int32),
        pltpu.VMEM((ROWS, D), jnp.float32),
        pltpu.VMEM((ROWS, D), jnp.float32),
        pltpu.VMEM((CH, D), jnp.float32),
        pltpu.SemaphoreType.DMA,
        pltpu.SemaphoreType.DMA,
    ],
)
def _sc_span_sum(xpad, idx, out, idx_v, rows_v0, rows_v1, emb_v, sem0, sem1):
    wid = lax.axis_index("s") * NC_SC + lax.axis_index("c")
    base = wid * SPW
    bufs = (rows_v0, rows_v1)
    sems = (sem0, sem1)

    pltpu.sync_copy(idx.at[pl.ds(base * MAX_W, SPW * MAX_W)], idx_v)

    def gather_start(ci, b):
        pltpu.async_copy(xpad.at[idx_v.at[pl.ds(ci * ROWS, ROWS)]], bufs[b], sems[b])

    def gather_wait(b):
        pltpu.make_async_copy(xpad.at[pl.ds(0, ROWS)], bufs[b], sems[b]).wait()

    gather_start(0, 0)
    gather_start(1, 1)

    def do_pair(ci2, carry):
        for b in range(2):
            ci = 2 * ci2 + b
            gather_wait(b)
            rows_v = bufs[b]

            def do_span(c, carry2):
                for j in range(D // 16):
                    sl = pl.ds(j * 16, 16)
                    emb_v[c, sl] = (rows_v[4 * c, sl] + rows_v[4 * c + 1, sl]
                                    + rows_v[4 * c + 2, sl] + rows_v[4 * c + 3, sl])
                return carry2

            lax.fori_loop(0, CH, do_span, 0)
            pltpu.sync_copy(emb_v, out.at[pl.ds(base + ci * CH, CH)])

            @pl.when(ci + 2 < NCHUNK)
            def _():
                gather_start(ci + 2, b)
        return carry

    lax.fori_loop(0, NCHUNK // 2, do_pair, 0)


def _ffn_ln(emb, wi_ref, bi_ref, wo_ref, bo_ref, g_ref, be_ref):
    inter = jnp.dot(emb, wi_ref[...], preferred_element_type=jnp.float32)
    inter = inter + bi_ref[...]
    inter = 0.5 * inter * (1.0 + lax.erf(inter * _SQRT_HALF))
    out = jnp.dot(inter, wo_ref[...], preferred_element_type=jnp.float32)
    out = out + bo_ref[...] + emb
    m = jnp.mean(out, axis=-1, keepdims=True)
    d = out - m
    v = jnp.mean(d * d, axis=-1, keepdims=True)
    return d * lax.rsqrt(v + 1e-12) * g_ref[...] + be_ref[...]


def _body(es_ref, iw_ref,
          wi_f_ref, bi_f_ref, wo_f_ref, bo_f_ref, g_f_ref, be_f_ref,
          wi_r_ref, bi_r_ref, wo_r_ref, bo_r_ref, g_r_ref, be_r_ref,
          wa_ref, ba_ref, wop_ref, bop_ref, out_ref):
    emb = es_ref[0] * iw_ref[0]
    y1 = _ffn_ln(emb, wi_f_ref, bi_f_ref, wo_f_ref, bo_f_ref, g_f_ref, be_f_ref)
    y2 = _ffn_ln(emb, wi_r_ref, bi_r_ref, wo_r_ref, bo_r_ref, g_r_ref, be_r_ref)
    c1 = jnp.dot(y1, wa_ref[...], preferred_element_type=jnp.float32) + ba_ref[...]
    c2 = jnp.dot(y2, wop_ref[...], preferred_element_type=jnp.float32) + bop_ref[...]
    out_ref[0] = jnp.concatenate([c1, c2], axis=-1)


@jax.jit
def _run(embsum, invw,
         Wi_f, bi_f, Wo_f, bo_f, g_f, be_f,
         Wi_r, bi_r, Wo_r, bo_r, g_r, be_r, Wa, ba, Wop, bop):
    const2 = pl.BlockSpec((1, D_FF), lambda i: (0, 0))
    constd = pl.BlockSpec((1, D), lambda i: (0, 0))
    w_big = pl.BlockSpec((D, D_FF), lambda i: (0, 0))
    w_big_t = pl.BlockSpec((D_FF, D), lambda i: (0, 0))
    w_cls = pl.BlockSpec((D, N_CLS), lambda i: (0, 0))
    b_cls = pl.BlockSpec((1, N_CLS), lambda i: (0, 0))
    out = pl.pallas_call(
        _body,
        grid=(G,),
        in_specs=[
            pl.BlockSpec((1, BN, D), lambda i: (i, 0, 0)),
            pl.BlockSpec((1, BN, 1), lambda i: (i, 0, 0)),
            w_big, const2, w_big_t, constd, constd, constd,
            w_big, const2, w_big_t, constd, constd, constd,
            w_cls, b_cls, w_cls, b_cls,
        ],
        out_specs=pl.BlockSpec((1, BN, 2 * N_CLS), lambda i: (i, 0, 0)),
        out_shape=jax.ShapeDtypeStruct((G, BN, 2 * N_CLS), jnp.float32),
        compiler_params=pltpu.CompilerParams(
            dimension_semantics=("arbitrary",),
            vmem_limit_bytes=120 * 1024 * 1024,
        ),
    )(embsum, invw,
      Wi_f, bi_f.reshape(1, D_FF), Wo_f, bo_f.reshape(1, D),
      g_f.reshape(1, D), be_f.reshape(1, D),
      Wi_r, bi_r.reshape(1, D_FF), Wo_r, bo_r.reshape(1, D),
      g_r.reshape(1, D), be_r.reshape(1, D),
      Wa, ba.reshape(1, N_CLS), Wop, bop.reshape(1, N_CLS))
    return out.reshape(B, SPAN_NUM, 2 * N_CLS)


def kernel(input_bert_features, attention_mask, spans, span_mask,
           related_spans_tensor, sentence_length,
           Wi_f, bi_f, Wo_f, bo_f, g_f, be_f,
           Wi_r, bi_r, Wo_r, bo_r, g_r, be_r, Wa, ba, Wop, bop):
    start = spans[..., 0]
    width = spans[..., 2]
    offs = jnp.arange(MAX_W, dtype=jnp.int32)
    brange = jnp.arange(B, dtype=jnp.int32) * S
    tok = jnp.clip(start[..., None] + offs, 0, S - 1) + brange[:, None, None]
    idx = jnp.where(offs < width[..., None], tok, ZROW)
    xpad = jnp.concatenate(
        [input_bert_features.reshape(B * S, D),
         jnp.zeros((8, D), jnp.float32)], axis=0)
    embsum = _sc_span_sum(xpad, idx.reshape(-1))

    invw = (1.0 / jnp.maximum(width.astype(jnp.float32), 1.0))
    invw = invw * span_mask.astype(jnp.float32)
    return _run(embsum.reshape(G, BN, D), invw.reshape(G, BN, 1),
                Wi_f, bi_f, Wo_f, bo_f, g_f, be_f,
                Wi_r, bi_r, Wo_r, bo_r, g_r, be_r, Wa, ba, Wop, bop)


# speedup vs baseline: 2.6613x; 2.6515x over previous
import functools

import jax
import jax.numpy as jnp
from jax import lax
from jax.experimental import pallas as pl
from jax.experimental.pallas import tpu as pltpu
from jax.experimental.pallas import tpu_sc as plsc

B, S, D = 8, 512, 768
SPAN_NUM = 2048
MAX_W = 4
D_FF = 3072
N_CLS = 3
BN = 512
G = (B * SPAN_NUM) // BN
_SQRT_HALF = 0.7071067811865476

NC_SC, NS_SC = 2, 16
NW = NC_SC * NS_SC
SPW = (B * SPAN_NUM) // NW
CH = 32
NIDX = 2
ROWS = NIDX * CH
NCHUNK = SPW // CH

_sc_mesh = plsc.VectorSubcoreMesh(core_axis_name="c", subcore_axis_name="s")


@functools.partial(
    pl.kernel,
    mesh=_sc_mesh,
    out_type=jax.ShapeDtypeStruct((B * SPAN_NUM, D), jnp.float32),
    scratch_types=[
        pltpu.VMEM((SPW * NIDX,), jnp.int32),
        pltpu.VMEM((ROWS, D), jnp.float32),
        pltpu.VMEM((ROWS, D), jnp.float32),
        pltpu.VMEM((CH, D), jnp.float32),
        pltpu.SemaphoreType.DMA,
        pltpu.SemaphoreType.DMA,
    ],
)
def _sc_span_sum(xpad, idx, out, idx_v, rows_v0, rows_v1, emb_v, sem0, sem1):
    wid = lax.axis_index("s") * NC_SC + lax.axis_index("c")
    base = wid * SPW
    bufs = (rows_v0, rows_v1)
    sems = (sem0, sem1)

    pltpu.sync_copy(idx.at[pl.ds(base * NIDX, SPW * NIDX)], idx_v)

    def gather_start(ci, b):
        pltpu.async_copy(xpad.at[idx_v.at[pl.ds(ci * ROWS, ROWS)]], bufs[b], sems[b])

    def gather_wait(b):
        pltpu.make_async_copy(xpad.at[pl.ds(0, ROWS)], bufs[b], sems[b]).wait()

    gather_start(0, 0)
    gather_start(1, 1)

    def do_pair(ci2, carry):
        for b in range(2):
            ci = 2 * ci2 + b
            gather_wait(b)
            rows_v = bufs[b]

            def do_span(c, carry2):
                for j in range(D // 16):
                    sl = pl.ds(j * 16, 16)
                    emb_v[c, sl] = rows_v[2 * c, sl] - rows_v[2 * c + 1, sl]
                return carry2

            lax.fori_loop(0, CH, do_span, 0)
            pltpu.sync_copy(emb_v, out.at[pl.ds(base + ci * CH, CH)])

            @pl.when(ci + 2 < NCHUNK)
            def _():
                gather_start(ci + 2, b)
        return carry

    lax.fori_loop(0, NCHUNK // 2, do_pair, 0)


def _prefix_body(x_ref, out_ref):
    i = lax.broadcasted_iota(jnp.int32, (S + 8, S), 0)
    t = lax.broadcasted_iota(jnp.int32, (S + 8, S), 1)
    m = (t < i).astype(jnp.float32)
    out_ref[0] = jnp.dot(m, x_ref[0], preferred_element_type=jnp.float32)


@jax.jit
def _prefix(x):
    return pl.pallas_call(
        _prefix_body,
        grid=(B,),
        in_specs=[pl.BlockSpec((1, S, D), lambda i: (i, 0, 0))],
        out_specs=pl.BlockSpec((1, S + 8, D), lambda i: (i, 0, 0)),
        out_shape=jax.ShapeDtypeStruct((B, S + 8, D), jnp.float32),
    )(x)


def _ffn_ln(emb, wi_ref, bi_ref, wo_ref, bo_ref, g_ref, be_ref):
    inter = jnp.dot(emb, wi_ref[...], preferred_element_type=jnp.float32)
    inter = inter + bi_ref[...]
    inter = 0.5 * inter * (1.0 + lax.erf(inter * _SQRT_HALF))
    out = jnp.dot(inter, wo_ref[...], preferred_element_type=jnp.float32)
    out = out + bo_ref[...] + emb
    m = jnp.mean(out, axis=-1, keepdims=True)
    d = out - m
    v = jnp.mean(d * d, axis=-1, keepdims=True)
    return d * lax.rsqrt(v + 1e-12) * g_ref[...] + be_ref[...]


def _body(es_ref, iw_ref,
          wi_f_ref, bi_f_ref, wo_f_ref, bo_f_ref, g_f_ref, be_f_ref,
          wi_r_ref, bi_r_ref, wo_r_ref, bo_r_ref, g_r_ref, be_r_ref,
          wa_ref, ba_ref, wop_ref, bop_ref, out_ref):
    emb = es_ref[0] * iw_ref[0]
    y1 = _ffn_ln(emb, wi_f_ref, bi_f_ref, wo_f_ref, bo_f_ref, g_f_ref, be_f_ref)
    y2 = _ffn_ln(emb, wi_r_ref, bi_r_ref, wo_r_ref, bo_r_ref, g_r_ref, be_r_ref)
    c1 = jnp.dot(y1, wa_ref[...], preferred_element_type=jnp.float32) + ba_ref[...]
    c2 = jnp.dot(y2, wop_ref[...], preferred_element_type=jnp.float32) + bop_ref[...]
    out_ref[0] = jnp.concatenate([c1, c2], axis=-1)


@jax.jit
def _run(embsum, invw,
         Wi_f, bi_f, Wo_f, bo_f, g_f, be_f,
         Wi_r, bi_r, Wo_r, bo_r, g_r, be_r, Wa, ba, Wop, bop):
    const2 = pl.BlockSpec((1, D_FF), lambda i: (0, 0))
    constd = pl.BlockSpec((1, D), lambda i: (0, 0))
    w_big = pl.BlockSpec((D, D_FF), lambda i: (0, 0))
    w_big_t = pl.BlockSpec((D_FF, D), lambda i: (0, 0))
    w_cls = pl.BlockSpec((D, N_CLS), lambda i: (0, 0))
    b_cls = pl.BlockSpec((1, N_CLS), lambda i: (0, 0))
    out = pl.pallas_call(
        _body,
        grid=(G,),
        in_specs=[
            pl.BlockSpec((1, BN, D), lambda i: (i, 0, 0)),
            pl.BlockSpec((1, BN, 1), lambda i: (i, 0, 0)),
            w_big, const2, w_big_t, constd, constd, constd,
            w_big, const2, w_big_t, constd, constd, constd,
            w_cls, b_cls, w_cls, b_cls,
        ],
        out_specs=pl.BlockSpec((1, BN, 2 * N_CLS), lambda i: (i, 0, 0)),
        out_shape=jax.ShapeDtypeStruct((G, BN, 2 * N_CLS), jnp.float32),
        compiler_params=pltpu.CompilerParams(
            dimension_semantics=("arbitrary",),
            vmem_limit_bytes=120 * 1024 * 1024,
        ),
    )(embsum, invw,
      Wi_f, bi_f.reshape(1, D_FF), Wo_f, bo_f.reshape(1, D),
      g_f.reshape(1, D), be_f.reshape(1, D),
      Wi_r, bi_r.reshape(1, D_FF), Wo_r, bo_r.reshape(1, D),
      g_r.reshape(1, D), be_r.reshape(1, D),
      Wa, ba.reshape(1, N_CLS), Wop, bop.reshape(1, N_CLS))
    return out.reshape(B, SPAN_NUM, 2 * N_CLS)


def kernel(input_bert_features, attention_mask, spans, span_mask,
           related_spans_tensor, sentence_length,
           Wi_f, bi_f, Wo_f, bo_f, g_f, be_f,
           Wi_r, bi_r, Wo_r, bo_r, g_r, be_r, Wa, ba, Wop, bop):
    start = spans[..., 0]
    width = spans[..., 2]
    brange = jnp.arange(B, dtype=jnp.int32) * (S + 8)
    hi = start + width + brange[:, None]
    lo = start + brange[:, None]
    idx = jnp.stack([hi, lo], axis=-1)
    ptab = _prefix(input_bert_features).reshape(B * (S + 8), D)
    embsum = _sc_span_sum(ptab, idx.reshape(-1))

    invw = (1.0 / jnp.maximum(width.astype(jnp.float32), 1.0))
    invw = invw * span_mask.astype(jnp.float32)
    return _run(embsum.reshape(G, BN, D), invw.reshape(G, BN, 1),
                Wi_f, bi_f, Wo_f, bo_f, g_f, be_f,
                Wi_r, bi_r, Wo_r, bo_r, g_r, be_r, Wa, ba, Wop, bop)


# speedup vs baseline: 2.8856x; 1.0843x over previous
import functools

import jax
import jax.numpy as jnp
from jax import lax
from jax.experimental import pallas as pl
from jax.experimental.pallas import tpu as pltpu
from jax.experimental.pallas import tpu_sc as plsc

B, S, D = 8, 512, 768
SPAN_NUM = 2048
MAX_W = 4
D_FF = 3072
N_CLS = 3
BN = 512
G = (B * SPAN_NUM) // BN
_SQRT_HALF = 0.7071067811865476

NC_SC, NS_SC = 2, 16
NW = NC_SC * NS_SC
SPW = (B * SPAN_NUM) // NW
CH = 32
NIDX = 2
ROWS = NIDX * CH
NCHUNK = SPW // CH
NSPLIT = 2
NPART = (B * SPAN_NUM) // NSPLIT
SPW_P = NPART // NW
NCHUNK_P = SPW_P // CH
G_P = NPART // BN

_sc_mesh = plsc.VectorSubcoreMesh(core_axis_name="c", subcore_axis_name="s")


@functools.partial(
    pl.kernel,
    mesh=_sc_mesh,
    out_type=jax.ShapeDtypeStruct((NPART, D), jnp.float32),
    scratch_types=[
        pltpu.VMEM((SPW_P * NIDX,), jnp.int32),
        pltpu.VMEM((ROWS, D), jnp.float32),
        pltpu.VMEM((ROWS, D), jnp.float32),
        pltpu.VMEM((CH, D), jnp.float32),
        pltpu.SemaphoreType.DMA,
        pltpu.SemaphoreType.DMA,
    ],
)
def _sc_span_sum(xpad, idx, out, idx_v, rows_v0, rows_v1, emb_v, sem0, sem1):
    wid = lax.axis_index("s") * NC_SC + lax.axis_index("c")
    base = wid * SPW_P
    bufs = (rows_v0, rows_v1)
    sems = (sem0, sem1)

    pltpu.sync_copy(idx.at[pl.ds(base * NIDX, SPW_P * NIDX)], idx_v)

    def gather_start(ci, b):
        pltpu.async_copy(xpad.at[idx_v.at[pl.ds(ci * ROWS, ROWS)]], bufs[b], sems[b])

    def gather_wait(b):
        pltpu.make_async_copy(xpad.at[pl.ds(0, ROWS)], bufs[b], sems[b]).wait()

    gather_start(0, 0)
    gather_start(1, 1)

    def do_pair(ci2, carry):
        for b in range(2):
            ci = 2 * ci2 + b
            gather_wait(b)
            rows_v = bufs[b]

            def do_span(c, carry2):
                for j in range(D // 16):
                    sl = pl.ds(j * 16, 16)
                    emb_v[c, sl] = rows_v[2 * c, sl] - rows_v[2 * c + 1, sl]
                return carry2

            lax.fori_loop(0, CH, do_span, 0)
            pltpu.sync_copy(emb_v, out.at[pl.ds(base + ci * CH, CH)])

            @pl.when(ci + 2 < NCHUNK_P)
            def _():
                gather_start(ci + 2, b)
        return carry

    lax.fori_loop(0, NCHUNK_P // 2, do_pair, 0)


def _prefix_body(x_ref, out_ref):
    i = lax.broadcasted_iota(jnp.int32, (S + 8, S), 0)
    t = lax.broadcasted_iota(jnp.int32, (S + 8, S), 1)
    m = (t < i).astype(jnp.float32)
    out_ref[0] = jnp.dot(m, x_ref[0], preferred_element_type=jnp.float32)


@jax.jit
def _prefix(x):
    return pl.pallas_call(
        _prefix_body,
        grid=(B,),
        in_specs=[pl.BlockSpec((1, S, D), lambda i: (i, 0, 0))],
        out_specs=pl.BlockSpec((1, S + 8, D), lambda i: (i, 0, 0)),
        out_shape=jax.ShapeDtypeStruct((B, S + 8, D), jnp.float32),
    )(x)


def _ffn_ln(emb, wi_ref, bi_ref, wo_ref, bo_ref, g_ref, be_ref):
    inter = jnp.dot(emb, wi_ref[...], preferred_element_type=jnp.float32)
    inter = inter + bi_ref[...]
    inter = 0.5 * inter * (1.0 + lax.erf(inter * _SQRT_HALF))
    out = jnp.dot(inter, wo_ref[...], preferred_element_type=jnp.float32)
    out = out + bo_ref[...] + emb
    m = jnp.mean(out, axis=-1, keepdims=True)
    d = out - m
    v = jnp.mean(d * d, axis=-1, keepdims=True)
    return d * lax.rsqrt(v + 1e-12) * g_ref[...] + be_ref[...]


def _body(es_ref, iw_ref,
          wi_f_ref, bi_f_ref, wo_f_ref, bo_f_ref, g_f_ref, be_f_ref,
          wi_r_ref, bi_r_ref, wo_r_ref, bo_r_ref, g_r_ref, be_r_ref,
          wa_ref, ba_ref, wop_ref, bop_ref, out_ref):
    emb = es_ref[0] * iw_ref[0]
    y1 = _ffn_ln(emb, wi_f_ref, bi_f_ref, wo_f_ref, bo_f_ref, g_f_ref, be_f_ref)
    y2 = _ffn_ln(emb, wi_r_ref, bi_r_ref, wo_r_ref, bo_r_ref, g_r_ref, be_r_ref)
    c1 = jnp.dot(y1, wa_ref[...], preferred_element_type=jnp.float32) + ba_ref[...]
    c2 = jnp.dot(y2, wop_ref[...], preferred_element_type=jnp.float32) + bop_ref[...]
    out_ref[0] = jnp.concatenate([c1, c2], axis=-1)


@jax.jit
def _run(embsum, invw,
         Wi_f, bi_f, Wo_f, bo_f, g_f, be_f,
         Wi_r, bi_r, Wo_r, bo_r, g_r, be_r, Wa, ba, Wop, bop):
    const2 = pl.BlockSpec((1, D_FF), lambda i: (0, 0))
    constd = pl.BlockSpec((1, D), lambda i: (0, 0))
    w_big = pl.BlockSpec((D, D_FF), lambda i: (0, 0))
    w_big_t = pl.BlockSpec((D_FF, D), lambda i: (0, 0))
    w_cls = pl.BlockSpec((D, N_CLS), lambda i: (0, 0))
    b_cls = pl.BlockSpec((1, N_CLS), lambda i: (0, 0))
    out = pl.pallas_call(
        _body,
        grid=(G_P,),
        in_specs=[
            pl.BlockSpec((1, BN, D), lambda i: (i, 0, 0)),
            pl.BlockSpec((1, BN, 1), lambda i: (i, 0, 0)),
            w_big, const2, w_big_t, constd, constd, constd,
            w_big, const2, w_big_t, constd, constd, constd,
            w_cls, b_cls, w_cls, b_cls,
        ],
        out_specs=pl.BlockSpec((1, BN, 2 * N_CLS), lambda i: (i, 0, 0)),
        out_shape=jax.ShapeDtypeStruct((G_P, BN, 2 * N_CLS), jnp.float32),
        compiler_params=pltpu.CompilerParams(
            dimension_semantics=("arbitrary",),
            vmem_limit_bytes=120 * 1024 * 1024,
        ),
    )(embsum, invw,
      Wi_f, bi_f.reshape(1, D_FF), Wo_f, bo_f.reshape(1, D),
      g_f.reshape(1, D), be_f.reshape(1, D),
      Wi_r, bi_r.reshape(1, D_FF), Wo_r, bo_r.reshape(1, D),
      g_r.reshape(1, D), be_r.reshape(1, D),
      Wa, ba.reshape(1, N_CLS), Wop, bop.reshape(1, N_CLS))
    return out.reshape(NPART, 2 * N_CLS)


def kernel(input_bert_features, attention_mask, spans, span_mask,
           related_spans_tensor, sentence_length,
           Wi_f, bi_f, Wo_f, bo_f, g_f, be_f,
           Wi_r, bi_r, Wo_r, bo_r, g_r, be_r, Wa, ba, Wop, bop):
    start = spans[..., 0]
    width = spans[..., 2]
    brange = jnp.arange(B, dtype=jnp.int32) * (S + 8)
    hi = start + width + brange[:, None]
    lo = start + brange[:, None]
    idx = jnp.stack([hi, lo], axis=-1)
    ptab = _prefix(input_bert_features).reshape(B * (S + 8), D)
    idxp = idx.reshape(NSPLIT, NPART * NIDX)
    invw = (1.0 / jnp.maximum(width.astype(jnp.float32), 1.0))
    invw = invw * span_mask.astype(jnp.float32)
    invwp = invw.reshape(NSPLIT, G_P, BN, 1)
    embs = [_sc_span_sum(ptab, idxp[p]) for p in range(NSPLIT)]
    outs = [_run(embs[p].reshape(G_P, BN, D), invwp[p],
                 Wi_f, bi_f, Wo_f, bo_f, g_f, be_f,
                 Wi_r, bi_r, Wo_r, bo_r, g_r, be_r, Wa, ba, Wop, bop)
            for p in range(NSPLIT)]
    return jnp.concatenate(outs, axis=0).reshape(B, SPAN_NUM, 2 * N_CLS)
